# baseline (speedup 1.0000x reference)
import jax
import jax.numpy as jnp
from jax.experimental import pallas as pl

S = 2048
HID = 1024
NH = 16
HD = 64
HD2 = 32
LAT = 128
FF = 512
NRE = 7
EPS = 1e-6

BT = 512
BQ = 256
BM = 512

NEG = jnp.finfo(jnp.float32).min
BF = jnp.bfloat16
F32 = jnp.float32


def _mm(a, b):
    return jax.lax.dot_general(a, b, (((1,), (0,)), ((), ())),
                               preferred_element_type=F32)


def _mmT(a, b):
    return jax.lax.dot_general(a, b, (((1,), (1,)), ((), ())),
                               preferred_element_type=F32)


def _rope_tables():
    inv_freq = 1.0 / (10000.0 ** (jnp.arange(0, HD2, 2, dtype=F32) / HD2))
    pos = jnp.arange(S, dtype=F32)
    ang = pos[:, None] * inv_freq[None, :]
    sin = jnp.sin(ang)
    cos = jnp.cos(ang)
    cos32 = jnp.concatenate([cos, cos], axis=-1)
    sin32 = jnp.concatenate([sin, sin], axis=-1)
    COS = jnp.tile(jnp.concatenate([jnp.ones((S, HD2)), cos32], axis=-1), (1, NH))
    SIN = jnp.tile(jnp.concatenate([jnp.zeros((S, HD2)), sin32], axis=-1), (1, NH))
    return COS, SIN


def _place_weights(W_qu, W_ropeq, W_ku, W_ropek):
    def rot(W):
        Wh = W.reshape(W.shape[0], NH, HD2)
        return jnp.concatenate([-Wh[:, :, HD2 // 2:], Wh[:, :, :HD2 // 2]], axis=-1)

    zq = jnp.zeros((LAT, NH, HD2))
    zk = jnp.zeros((HID, NH, HD2))
    c = 0.125 * 1.4426950408889634
    Aq = c * jnp.concatenate([W_qu.reshape(LAT, NH, HD2),
                              W_ropeq.reshape(LAT, NH, HD2)], axis=-1).reshape(LAT, NH * HD)
    Bq = c * jnp.concatenate([zq, rot(W_ropeq)], axis=-1).reshape(LAT, NH * HD)
    Ak2 = jnp.concatenate([W_ku.reshape(LAT, NH, HD2), zq], axis=-1).reshape(LAT, NH * HD)
    Akr = jnp.concatenate([zk, W_ropek.reshape(HID, NH, HD2)], axis=-1).reshape(HID, NH * HD)
    Bk = jnp.concatenate([zk, rot(W_ropek)], axis=-1).reshape(HID, NH * HD)
    return Aq, Bq, Ak2, Akr, Bk


def _qkv_kernel(x_ref, ln1_ref, Wqd_ref, Wkvd_ref, Aq_ref, Bq_ref, Ak2_ref,
                Akr_ref, Bk_ref, Wvu_ref, cos_ref, sin_ref,
                q_ref, k_ref, v_ref):
    x = x_ref[...]
    h = (x * jax.lax.rsqrt(jnp.mean(x * x, axis=-1, keepdims=True) + EPS)
         * ln1_ref[...]).astype(BF)
    qd = _mm(h, Wqd_ref[...]).astype(BF)
    kvd = _mm(h, Wkvd_ref[...]).astype(BF)
    cos = cos_ref[...]
    sin = sin_ref[...]
    q_ref[...] = (_mm(qd, Aq_ref[...]) * cos + _mm(qd, Bq_ref[...]) * sin).astype(BF)
    k_ref[...] = (_mm(kvd, Ak2_ref[...]) + _mm(h, Akr_ref[...]) * cos
                  + _mm(h, Bk_ref[...]) * sin).astype(BF)
    v_ref[...] = _mm(kvd, Wvu_ref[...]).astype(BF)


def _attn_kernel(q_ref, k_ref, v_ref, o_ref):
    t = pl.program_id(0)
    qv = q_ref[...]

    def chunk(j, acc, den, masked):
        kj = k_ref[pl.ds(j * BQ, BQ), :]
        vj = v_ref[pl.ds(j * BQ, BQ), :]
        accs = []
        dens = []
        if masked:
            r = jax.lax.broadcasted_iota(jnp.int32, (BQ, BQ), 0)
            c = jax.lax.broadcasted_iota(jnp.int32, (BQ, BQ), 1)
            dmask = c <= r
        for h in range(NH):
            sl = slice(h * HD, (h + 1) * HD)
            p = jnp.exp2(_mmT(qv[:, sl], kj[:, sl]))
            if masked:
                p = jnp.where(dmask, p, 0.0)
            accs.append(_mm(p.astype(BF), vj[:, sl]))
            dens.append(jnp.sum(p, axis=-1, keepdims=True))
        return acc + jnp.concatenate(accs, axis=-1), den + jnp.concatenate(dens, axis=-1)

    def body(j, carry):
        acc, den = carry
        return chunk(j, acc, den, masked=False)

    acc0 = jnp.zeros((BQ, NH * HD), F32)
    den0 = jnp.zeros((BQ, NH), F32)
    acc, den = jax.lax.fori_loop(0, t, body, (acc0, den0))
    acc, den = chunk(t, acc, den, masked=True)
    recip = 1.0 / den
    for h in range(NH):
        sl = slice(h * HD, (h + 1) * HD)
        o_ref[:, sl] = (acc[:, sl] * recip[:, h:h + 1]).astype(BF)


def _post_kernel(y_ref, x_ref, ln2_ref, Wo_ref, Wgs_ref, Wus_ref, Wds_ref,
                 Wr_ref, rb_ref, acc0_ref, z_ref, w_ref):
    x2 = x_ref[...] + _mm(y_ref[...], Wo_ref[...])
    z = x2 * jax.lax.rsqrt(jnp.mean(x2 * x2, axis=-1, keepdims=True) + EPS) * ln2_ref[...]
    zb = z.astype(BF)
    z_ref[...] = zb
    g = _mm(zb, Wgs_ref[...])
    u = _mm(zb, Wus_ref[...])
    acc0_ref[...] = x2 + _mm((g * jax.nn.sigmoid(g) * u).astype(BF), Wds_ref[...])
    logits = _mm(z, Wr_ref[...]) + rb_ref[...]
    probs = jax.nn.sigmoid(logits)
    iota = jax.lax.broadcasted_iota(jnp.int32, (BT, 8), 1)
    v1 = jnp.max(probs, axis=-1, keepdims=True)
    i1 = jnp.min(jnp.where(probs == v1, iota, 8), axis=-1, keepdims=True)
    oh1 = iota == i1
    masked = jnp.where(oh1, -1.0, probs)
    v2 = jnp.max(masked, axis=-1, keepdims=True)
    i2 = jnp.min(jnp.where(masked == v2, iota, 8), axis=-1, keepdims=True)
    oh2 = iota == i2
    w_ref[...] = (oh1 * v1 + oh2 * v2) / (v1 + v2)


def _moe_kernel(z_ref, acc0_ref, w_ref, Wg_ref, Wu_ref, Wd_ref, out_ref):
    z = z_ref[...]
    wv = w_ref[...]
    out = acc0_ref[...]
    for e in range(NRE):
        g = _mm(z, Wg_ref[e]).astype(BF)
        u = _mm(z, Wu_ref[e]).astype(BF)
        o = _mm(g * jax.nn.sigmoid(g) * u, Wd_ref[e])
        out += o * wv[:, e:e + 1]
    out_ref[...] = out


def kernel(x, ln1_w, ln2_w, W_kvd, W_qd, W_ku, W_qu, W_vu, W_ropek, W_ropeq,
           W_o, Wr, r_bias, Wg_s, Wu_s, Wd_s, Wg_r, Wu_r, Wd_r):
    x2d = x[0]
    COS, SIN = _rope_tables()
    Aq, Bq, Ak2, Akr, Bk = _place_weights(W_qu, W_ropeq, W_ku, W_ropek)
    ln1 = ln1_w[None]
    ln2 = ln2_w[None]
    Wr_pad = jnp.pad(Wr, ((0, 0), (0, 1)))
    rb_pad = jnp.pad(r_bias, (0, 1), constant_values=-1e30)[None]
    bf = lambda a: a.astype(BF)

    full = lambda shape: pl.BlockSpec(shape, lambda t: (0,) * len(shape))

    q_cat, k_cat, v_cat = pl.pallas_call(
        _qkv_kernel,
        grid=(S // BT,),
        in_specs=[
            pl.BlockSpec((BT, HID), lambda t: (t, 0)),
            full((1, HID)),
            full((HID, LAT)), full((HID, LAT)),
            full((LAT, NH * HD)), full((LAT, NH * HD)), full((LAT, NH * HD)),
            full((HID, NH * HD)), full((HID, NH * HD)),
            full((LAT, NH * HD)),
            pl.BlockSpec((BT, NH * HD), lambda t: (t, 0)),
            pl.BlockSpec((BT, NH * HD), lambda t: (t, 0)),
        ],
        out_specs=[
            pl.BlockSpec((BT, NH * HD), lambda t: (t, 0)),
            pl.BlockSpec((BT, NH * HD), lambda t: (t, 0)),
            pl.BlockSpec((BT, NH * HD), lambda t: (t, 0)),
        ],
        out_shape=[
            jax.ShapeDtypeStruct((S, NH * HD), BF),
            jax.ShapeDtypeStruct((S, NH * HD), BF),
            jax.ShapeDtypeStruct((S, NH * HD), BF),
        ],
    )(x2d, ln1, bf(W_qd), bf(W_kvd), bf(Aq), bf(Bq), bf(Ak2), bf(Akr), bf(Bk),
      bf(W_vu), COS, SIN)

    y = pl.pallas_call(
        _attn_kernel,
        grid=(S // BQ,),
        in_specs=[
            pl.BlockSpec((BQ, NH * HD), lambda t: (t, 0)),
            full((S, NH * HD)),
            full((S, NH * HD)),
        ],
        out_specs=pl.BlockSpec((BQ, NH * HD), lambda t: (t, 0)),
        out_shape=jax.ShapeDtypeStruct((S, NH * HD), BF),
    )(q_cat, k_cat, v_cat)

    acc0, z, w = pl.pallas_call(
        _post_kernel,
        grid=(S // BT,),
        in_specs=[
            pl.BlockSpec((BT, HID), lambda t: (t, 0)),
            pl.BlockSpec((BT, HID), lambda t: (t, 0)),
            full((1, HID)),
            full((HID, HID)),
            full((HID, FF)), full((HID, FF)), full((FF, HID)),
            full((HID, 8)),
            full((1, 8)),
        ],
        out_specs=[
            pl.BlockSpec((BT, HID), lambda t: (t, 0)),
            pl.BlockSpec((BT, HID), lambda t: (t, 0)),
            pl.BlockSpec((BT, 8), lambda t: (t, 0)),
        ],
        out_shape=[
            jax.ShapeDtypeStruct((S, HID), F32),
            jax.ShapeDtypeStruct((S, HID), BF),
            jax.ShapeDtypeStruct((S, 8), F32),
        ],
    )(y, x2d, ln2, bf(W_o), bf(Wg_s), bf(Wu_s), bf(Wd_s), Wr_pad, rb_pad)

    out = pl.pallas_call(
        _moe_kernel,
        grid=(S // BM,),
        in_specs=[
            pl.BlockSpec((BM, HID), lambda t: (t, 0)),
            pl.BlockSpec((BM, HID), lambda t: (t, 0)),
            pl.BlockSpec((BM, 8), lambda t: (t, 0)),
            full((NRE, HID, FF)),
            full((NRE, HID, FF)),
            full((NRE, FF, HID)),
        ],
        out_specs=pl.BlockSpec((BM, HID), lambda t: (t, 0)),
        out_shape=jax.ShapeDtypeStruct((S, HID), F32),
    )(z, acc0, w, bf(Wg_r), bf(Wu_r), bf(Wd_r))

    return out[None]


# speedup vs baseline: 1.0375x; 1.0375x over previous
import jax
import jax.numpy as jnp
from jax.experimental import pallas as pl

S = 2048
HID = 1024
NH = 16
HD = 64
HD2 = 32
LAT = 128
FF = 512
NRE = 7
EPS = 1e-6

BT = 512
BQ = 256
BM = 512

NEG = jnp.finfo(jnp.float32).min
BF = jnp.bfloat16
F32 = jnp.float32


def _mm(a, b):
    return jax.lax.dot_general(a, b, (((1,), (0,)), ((), ())),
                               preferred_element_type=F32)


def _mmT(a, b):
    return jax.lax.dot_general(a, b, (((1,), (1,)), ((), ())),
                               preferred_element_type=F32)


def _rope_tables():
    inv_freq = 1.0 / (10000.0 ** (jnp.arange(0, HD2, 2, dtype=F32) / HD2))
    pos = jnp.arange(S, dtype=F32)
    ang = pos[:, None] * inv_freq[None, :]
    sin = jnp.sin(ang)
    cos = jnp.cos(ang)
    cos32 = jnp.concatenate([cos, cos], axis=-1)
    sin32 = jnp.concatenate([sin, sin], axis=-1)
    COS = jnp.tile(jnp.concatenate([jnp.ones((S, HD2)), cos32], axis=-1), (1, NH))
    SIN = jnp.tile(jnp.concatenate([jnp.zeros((S, HD2)), sin32], axis=-1), (1, NH))
    return COS, SIN


def _place_weights(W_qu, W_ropeq, W_ku, W_ropek):
    def rot(W):
        Wh = W.reshape(W.shape[0], NH, HD2)
        return jnp.concatenate([-Wh[:, :, HD2 // 2:], Wh[:, :, :HD2 // 2]], axis=-1)

    zq = jnp.zeros((LAT, NH, HD2))
    zk = jnp.zeros((HID, NH, HD2))
    c = 0.125 * 1.4426950408889634
    Aq = c * jnp.concatenate([W_qu.reshape(LAT, NH, HD2),
                              W_ropeq.reshape(LAT, NH, HD2)], axis=-1).reshape(LAT, NH * HD)
    Bq = c * jnp.concatenate([zq, rot(W_ropeq)], axis=-1).reshape(LAT, NH * HD)
    Ak2 = jnp.concatenate([W_ku.reshape(LAT, NH, HD2), zq], axis=-1).reshape(LAT, NH * HD)
    Akr = jnp.concatenate([zk, W_ropek.reshape(HID, NH, HD2)], axis=-1).reshape(HID, NH * HD)
    Bk = jnp.concatenate([zk, rot(W_ropek)], axis=-1).reshape(HID, NH * HD)
    return Aq, Bq, Ak2, Akr, Bk


def _qkv_kernel(x_ref, ln1_ref, Wqd_ref, Wkvd_ref, Aq_ref, Bq_ref, Ak2_ref,
                Akr_ref, Bk_ref, Wvu_ref, cos_ref, sin_ref,
                q_ref, k_ref, v_ref):
    x = x_ref[...]
    h = (x * jax.lax.rsqrt(jnp.mean(x * x, axis=-1, keepdims=True) + EPS)
         * ln1_ref[...]).astype(BF)
    qd = _mm(h, Wqd_ref[...]).astype(BF)
    kvd = _mm(h, Wkvd_ref[...]).astype(BF)
    cos = cos_ref[...]
    sin = sin_ref[...]
    q_ref[...] = (_mm(qd, Aq_ref[...]) * cos + _mm(qd, Bq_ref[...]) * sin).astype(BF)
    k_ref[...] = (_mm(kvd, Ak2_ref[...]) + _mm(h, Akr_ref[...]) * cos
                  + _mm(h, Bk_ref[...]) * sin).astype(BF)
    v_ref[...] = _mm(kvd, Wvu_ref[...]).astype(BF)


def _attn_kernel(q_ref, k_ref, v_ref, o_ref):
    t = pl.program_id(0)
    qv = q_ref[...]

    def chunk(j, acc, den, masked):
        kj = k_ref[pl.ds(j * BQ, BQ), :]
        vj = v_ref[pl.ds(j * BQ, BQ), :]
        accs = []
        dens = []
        if masked:
            r = jax.lax.broadcasted_iota(jnp.int32, (BQ, BQ), 0)
            c = jax.lax.broadcasted_iota(jnp.int32, (BQ, BQ), 1)
            dmask = c <= r
        for h in range(NH):
            sl = slice(h * HD, (h + 1) * HD)
            p = jnp.exp2(_mmT(qv[:, sl], kj[:, sl]))
            if masked:
                p = jnp.where(dmask, p, 0.0)
            accs.append(_mm(p.astype(BF), vj[:, sl]))
            dens.append(jnp.sum(p, axis=-1, keepdims=True))
        return acc + jnp.concatenate(accs, axis=-1), den + jnp.concatenate(dens, axis=-1)

    def body(j, carry):
        acc, den = carry
        return chunk(j, acc, den, masked=False)

    acc0 = jnp.zeros((BQ, NH * HD), F32)
    den0 = jnp.zeros((BQ, NH), F32)
    acc, den = jax.lax.fori_loop(0, t, body, (acc0, den0))
    acc, den = chunk(t, acc, den, masked=True)
    recip = 1.0 / den
    for h in range(NH):
        sl = slice(h * HD, (h + 1) * HD)
        o_ref[:, sl] = (acc[:, sl] * recip[:, h:h + 1]).astype(BF)


def _post_moe_kernel(y_ref, x_ref, ln2_ref, Wo_ref, Wgs_ref, Wus_ref,
                     Wds_ref, Wr_ref, rb_ref, Wg_ref, Wu_ref, Wd_ref, out_ref):
    x2 = x_ref[...] + _mm(y_ref[...], Wo_ref[...])
    z = x2 * jax.lax.rsqrt(jnp.mean(x2 * x2, axis=-1, keepdims=True) + EPS) * ln2_ref[...]
    zb = z.astype(BF)
    g = _mm(zb, Wgs_ref[...])
    u = _mm(zb, Wus_ref[...])
    out = x2 + _mm((g * jax.nn.sigmoid(g) * u).astype(BF), Wds_ref[...])
    logits = _mm(z, Wr_ref[...]) + rb_ref[...]
    probs = jax.nn.sigmoid(logits)
    iota = jax.lax.broadcasted_iota(jnp.int32, (BT, 8), 1)
    v1 = jnp.max(probs, axis=-1, keepdims=True)
    i1 = jnp.min(jnp.where(probs == v1, iota, 8), axis=-1, keepdims=True)
    oh1 = iota == i1
    masked = jnp.where(oh1, -1.0, probs)
    v2 = jnp.max(masked, axis=-1, keepdims=True)
    i2 = jnp.min(jnp.where(masked == v2, iota, 8), axis=-1, keepdims=True)
    oh2 = iota == i2
    w = (oh1 * v1 + oh2 * v2) / (v1 + v2)
    for e in range(NRE):
        g = _mm(zb, Wg_ref[e])
        u = _mm(zb, Wu_ref[e])
        o = _mm((g * jax.nn.sigmoid(g) * u).astype(BF), Wd_ref[e])
        out += o * w[:, e:e + 1]
    out_ref[...] = out


def kernel(x, ln1_w, ln2_w, W_kvd, W_qd, W_ku, W_qu, W_vu, W_ropek, W_ropeq,
           W_o, Wr, r_bias, Wg_s, Wu_s, Wd_s, Wg_r, Wu_r, Wd_r):
    x2d = x[0]
    COS, SIN = _rope_tables()
    Aq, Bq, Ak2, Akr, Bk = _place_weights(W_qu, W_ropeq, W_ku, W_ropek)
    ln1 = ln1_w[None]
    ln2 = ln2_w[None]
    Wr_pad = jnp.pad(Wr, ((0, 0), (0, 1)))
    rb_pad = jnp.pad(r_bias, (0, 1), constant_values=-1e30)[None]
    bf = lambda a: a.astype(BF)

    full = lambda shape: pl.BlockSpec(shape, lambda t: (0,) * len(shape))

    q_cat, k_cat, v_cat = pl.pallas_call(
        _qkv_kernel,
        grid=(S // BT,),
        in_specs=[
            pl.BlockSpec((BT, HID), lambda t: (t, 0)),
            full((1, HID)),
            full((HID, LAT)), full((HID, LAT)),
            full((LAT, NH * HD)), full((LAT, NH * HD)), full((LAT, NH * HD)),
            full((HID, NH * HD)), full((HID, NH * HD)),
            full((LAT, NH * HD)),
            pl.BlockSpec((BT, NH * HD), lambda t: (t, 0)),
            pl.BlockSpec((BT, NH * HD), lambda t: (t, 0)),
        ],
        out_specs=[
            pl.BlockSpec((BT, NH * HD), lambda t: (t, 0)),
            pl.BlockSpec((BT, NH * HD), lambda t: (t, 0)),
            pl.BlockSpec((BT, NH * HD), lambda t: (t, 0)),
        ],
        out_shape=[
            jax.ShapeDtypeStruct((S, NH * HD), BF),
            jax.ShapeDtypeStruct((S, NH * HD), BF),
            jax.ShapeDtypeStruct((S, NH * HD), BF),
        ],
    )(x2d, ln1, bf(W_qd), bf(W_kvd), bf(Aq), bf(Bq), bf(Ak2), bf(Akr), bf(Bk),
      bf(W_vu), COS, SIN)

    y = pl.pallas_call(
        _attn_kernel,
        grid=(S // BQ,),
        in_specs=[
            pl.BlockSpec((BQ, NH * HD), lambda t: (t, 0)),
            full((S, NH * HD)),
            full((S, NH * HD)),
        ],
        out_specs=pl.BlockSpec((BQ, NH * HD), lambda t: (t, 0)),
        out_shape=jax.ShapeDtypeStruct((S, NH * HD), BF),
    )(q_cat, k_cat, v_cat)

    out = pl.pallas_call(
        _post_moe_kernel,
        grid=(S // BT,),
        in_specs=[
            pl.BlockSpec((BT, HID), lambda t: (t, 0)),
            pl.BlockSpec((BT, HID), lambda t: (t, 0)),
            full((1, HID)),
            full((HID, HID)),
            full((HID, FF)), full((HID, FF)), full((FF, HID)),
            full((HID, 8)),
            full((1, 8)),
            full((NRE, HID, FF)),
            full((NRE, HID, FF)),
            full((NRE, FF, HID)),
        ],
        out_specs=pl.BlockSpec((BT, HID), lambda t: (t, 0)),
        out_shape=jax.ShapeDtypeStruct((S, HID), F32),
    )(y, x2d, ln2, bf(W_o), bf(Wg_s), bf(Wu_s), bf(Wd_s), Wr_pad, rb_pad,
      bf(Wg_r), bf(Wu_r), bf(Wd_r))

    return out[None]


# speedup vs baseline: 1.1406x; 1.0994x over previous
import jax
import jax.numpy as jnp
from jax.experimental import pallas as pl

S = 2048
HID = 1024
NH = 16
HD = 64
HD2 = 32
LAT = 128
FF = 512
NRE = 7
EPS = 1e-6

BT = 512
BQ = 512
BM = 512

NEG = jnp.finfo(jnp.float32).min
BF = jnp.bfloat16
F32 = jnp.float32


def _mm(a, b):
    return jax.lax.dot_general(a, b, (((1,), (0,)), ((), ())),
                               preferred_element_type=F32)


def _mmT(a, b):
    return jax.lax.dot_general(a, b, (((1,), (1,)), ((), ())),
                               preferred_element_type=F32)


def _rope_tables():
    inv_freq = 1.0 / (10000.0 ** (jnp.arange(0, HD2, 2, dtype=F32) / HD2))
    pos = jnp.arange(S, dtype=F32)
    ang = pos[:, None] * inv_freq[None, :]
    sin = jnp.sin(ang)
    cos = jnp.cos(ang)
    cos32 = jnp.concatenate([cos, cos], axis=-1)
    sin32 = jnp.concatenate([sin, sin], axis=-1)
    COS = jnp.tile(jnp.concatenate([jnp.ones((S, HD2)), cos32], axis=-1), (1, NH))
    SIN = jnp.tile(jnp.concatenate([jnp.zeros((S, HD2)), sin32], axis=-1), (1, NH))
    return COS, SIN


def _place_weights(W_qu, W_ropeq, W_ku, W_ropek):
    def rot(W):
        Wh = W.reshape(W.shape[0], NH, HD2)
        return jnp.concatenate([-Wh[:, :, HD2 // 2:], Wh[:, :, :HD2 // 2]], axis=-1)

    zq = jnp.zeros((LAT, NH, HD2))
    zk = jnp.zeros((HID, NH, HD2))
    c = 0.125 * 1.4426950408889634
    Aq = c * jnp.concatenate([W_qu.reshape(LAT, NH, HD2),
                              W_ropeq.reshape(LAT, NH, HD2)], axis=-1).reshape(LAT, NH * HD)
    Bq = c * jnp.concatenate([zq, rot(W_ropeq)], axis=-1).reshape(LAT, NH * HD)
    Ak2 = jnp.concatenate([W_ku.reshape(LAT, NH, HD2), zq], axis=-1).reshape(LAT, NH * HD)
    Akr = jnp.concatenate([zk, W_ropek.reshape(HID, NH, HD2)], axis=-1).reshape(HID, NH * HD)
    Bk = jnp.concatenate([zk, rot(W_ropek)], axis=-1).reshape(HID, NH * HD)
    return Aq, Bq, Ak2, Akr, Bk


def _qkv_kernel(x_ref, ln1_ref, Wqd_ref, Wkvd_ref, Aq_ref, Bq_ref, Ak2_ref,
                Akr_ref, Bk_ref, Wvu_ref, cos_ref, sin_ref,
                q_ref, k_ref, v_ref):
    x = x_ref[...]
    h = (x * jax.lax.rsqrt(jnp.mean(x * x, axis=-1, keepdims=True) + EPS)
         * ln1_ref[...]).astype(BF)
    qd = _mm(h, Wqd_ref[...]).astype(BF)
    kvd = _mm(h, Wkvd_ref[...]).astype(BF)
    cos = cos_ref[...]
    sin = sin_ref[...]
    q_ref[...] = (_mm(qd, Aq_ref[...]) * cos + _mm(qd, Bq_ref[...]) * sin).astype(BF)
    k_ref[...] = (_mm(kvd, Ak2_ref[...]) + _mm(h, Akr_ref[...]) * cos
                  + _mm(h, Bk_ref[...]) * sin).astype(BF)
    v_ref[...] = _mm(kvd, Wvu_ref[...]).astype(BF)


def _attn_kernel(q_ref, k_ref, v_ref, o_ref):
    t = pl.program_id(0)
    qv = q_ref[...]

    def chunk(j, acc, den, masked):
        kj = k_ref[pl.ds(j * BQ, BQ), :]
        vj = v_ref[pl.ds(j * BQ, BQ), :]
        accs = []
        dens = []
        if masked:
            r = jax.lax.broadcasted_iota(jnp.int32, (BQ, BQ), 0)
            c = jax.lax.broadcasted_iota(jnp.int32, (BQ, BQ), 1)
            dmask = c <= r
        for h in range(NH):
            sl = slice(h * HD, (h + 1) * HD)
            p = jnp.exp2(_mmT(qv[:, sl], kj[:, sl]))
            if masked:
                p = jnp.where(dmask, p, 0.0)
            accs.append(_mm(p.astype(BF), vj[:, sl]))
            dens.append(jnp.sum(p, axis=-1, keepdims=True))
        return acc + jnp.concatenate(accs, axis=-1), den + jnp.concatenate(dens, axis=-1)

    def body(j, carry):
        acc, den = carry
        return chunk(j, acc, den, masked=False)

    acc0 = jnp.zeros((BQ, NH * HD), F32)
    den0 = jnp.zeros((BQ, NH), F32)
    acc, den = jax.lax.fori_loop(0, t, body, (acc0, den0))
    acc, den = chunk(t, acc, den, masked=True)
    recip = 1.0 / den
    for h in range(NH):
        sl = slice(h * HD, (h + 1) * HD)
        o_ref[:, sl] = (acc[:, sl] * recip[:, h:h + 1]).astype(BF)


def _post_moe_kernel(y_ref, x_ref, ln2_ref, Wo_ref, Wgs_ref, Wus_ref,
                     Wds_ref, Wr_ref, rb_ref, Wg_ref, Wu_ref, Wd_ref, out_ref):
    x2 = x_ref[...] + _mm(y_ref[...], Wo_ref[...])
    z = x2 * jax.lax.rsqrt(jnp.mean(x2 * x2, axis=-1, keepdims=True) + EPS) * ln2_ref[...]
    zb = z.astype(BF)
    g = _mm(zb, Wgs_ref[...])
    u = _mm(zb, Wus_ref[...])
    out = x2 + _mm((g * jax.nn.sigmoid(g) * u).astype(BF), Wds_ref[...])
    logits = _mm(z, Wr_ref[...]) + rb_ref[...]
    probs = jax.nn.sigmoid(logits)
    iota = jax.lax.broadcasted_iota(jnp.int32, (BT, 8), 1)
    v1 = jnp.max(probs, axis=-1, keepdims=True)
    i1 = jnp.min(jnp.where(probs == v1, iota, 8), axis=-1, keepdims=True)
    oh1 = iota == i1
    masked = jnp.where(oh1, -1.0, probs)
    v2 = jnp.max(masked, axis=-1, keepdims=True)
    i2 = jnp.min(jnp.where(masked == v2, iota, 8), axis=-1, keepdims=True)
    oh2 = iota == i2
    w = (oh1 * v1 + oh2 * v2) / (v1 + v2)
    for e in range(NRE):
        g = _mm(zb, Wg_ref[e])
        u = _mm(zb, Wu_ref[e])
        o = _mm((g * jax.nn.sigmoid(g) * u).astype(BF), Wd_ref[e])
        out += o * w[:, e:e + 1]
    out_ref[...] = out


def kernel(x, ln1_w, ln2_w, W_kvd, W_qd, W_ku, W_qu, W_vu, W_ropek, W_ropeq,
           W_o, Wr, r_bias, Wg_s, Wu_s, Wd_s, Wg_r, Wu_r, Wd_r):
    x2d = x[0]
    COS, SIN = _rope_tables()
    Aq, Bq, Ak2, Akr, Bk = _place_weights(W_qu, W_ropeq, W_ku, W_ropek)
    ln1 = ln1_w[None]
    ln2 = ln2_w[None]
    Wr_pad = jnp.pad(Wr, ((0, 0), (0, 1)))
    rb_pad = jnp.pad(r_bias, (0, 1), constant_values=-1e30)[None]
    bf = lambda a: a.astype(BF)

    full = lambda shape: pl.BlockSpec(shape, lambda t: (0,) * len(shape))

    q_cat, k_cat, v_cat = pl.pallas_call(
        _qkv_kernel,
        grid=(S // BT,),
        in_specs=[
            pl.BlockSpec((BT, HID), lambda t: (t, 0)),
            full((1, HID)),
            full((HID, LAT)), full((HID, LAT)),
            full((LAT, NH * HD)), full((LAT, NH * HD)), full((LAT, NH * HD)),
            full((HID, NH * HD)), full((HID, NH * HD)),
            full((LAT, NH * HD)),
            pl.BlockSpec((BT, NH * HD), lambda t: (t, 0)),
            pl.BlockSpec((BT, NH * HD), lambda t: (t, 0)),
        ],
        out_specs=[
            pl.BlockSpec((BT, NH * HD), lambda t: (t, 0)),
            pl.BlockSpec((BT, NH * HD), lambda t: (t, 0)),
            pl.BlockSpec((BT, NH * HD), lambda t: (t, 0)),
        ],
        out_shape=[
            jax.ShapeDtypeStruct((S, NH * HD), BF),
            jax.ShapeDtypeStruct((S, NH * HD), BF),
            jax.ShapeDtypeStruct((S, NH * HD), BF),
        ],
    )(x2d, ln1, bf(W_qd), bf(W_kvd), bf(Aq), bf(Bq), bf(Ak2), bf(Akr), bf(Bk),
      bf(W_vu), COS, SIN)

    y = pl.pallas_call(
        _attn_kernel,
        grid=(S // BQ,),
        in_specs=[
            pl.BlockSpec((BQ, NH * HD), lambda t: (t, 0)),
            full((S, NH * HD)),
            full((S, NH * HD)),
        ],
        out_specs=pl.BlockSpec((BQ, NH * HD), lambda t: (t, 0)),
        out_shape=jax.ShapeDtypeStruct((S, NH * HD), BF),
    )(q_cat, k_cat, v_cat)

    out = pl.pallas_call(
        _post_moe_kernel,
        grid=(S // BT,),
        in_specs=[
            pl.BlockSpec((BT, HID), lambda t: (t, 0)),
            pl.BlockSpec((BT, HID), lambda t: (t, 0)),
            full((1, HID)),
            full((HID, HID)),
            full((HID, FF)), full((HID, FF)), full((FF, HID)),
            full((HID, 8)),
            full((1, 8)),
            full((NRE, HID, FF)),
            full((NRE, HID, FF)),
            full((NRE, FF, HID)),
        ],
        out_specs=pl.BlockSpec((BT, HID), lambda t: (t, 0)),
        out_shape=jax.ShapeDtypeStruct((S, HID), F32),
    )(y, x2d, ln2, bf(W_o), bf(Wg_s), bf(Wu_s), bf(Wd_s), Wr_pad, rb_pad,
      bf(Wg_r), bf(Wu_r), bf(Wd_r))

    return out[None]
